# baseline (speedup 1.0000x reference)
import functools

import jax
import jax.numpy as jnp
from jax import lax
from jax.experimental import pallas as pl
from jax.experimental.pallas import tpu as pltpu
from jax.experimental.pallas import tpu_sc as plsc

_B, _N, _CIN, _COUT, _K = 4, 8192, 128, 128, 16
_M = _N // 4
_EPS = 1e-5

_LANES = 128
_CS = 4096
_NCH = _N // _CS


def _fps_body(px_ref, py_ref, pz_ref, ox_ref, oy_ref, oz_ref, dist_ref):
    lanes = lax.broadcasted_iota(jnp.int32, (_B, _N), 1)
    cols = lax.broadcasted_iota(jnp.int32, (_B, _M), 1)
    px = px_ref[...]
    py = py_ref[...]
    pz = pz_ref[...]
    dist_ref[...] = jnp.full((_B, _N), jnp.inf, jnp.float32)
    ox_ref[...] = jnp.zeros((_B, _M), jnp.float32)
    oy_ref[...] = jnp.zeros((_B, _M), jnp.float32)
    oz_ref[...] = jnp.zeros((_B, _M), jnp.float32)

    def body(i, cur):
        msk = lanes == cur
        lx = jnp.max(jnp.where(msk, px, -jnp.inf), axis=1, keepdims=True)
        ly = jnp.max(jnp.where(msk, py, -jnp.inf), axis=1, keepdims=True)
        lz = jnp.max(jnp.where(msk, pz, -jnp.inf), axis=1, keepdims=True)
        sel = cols == i
        ox_ref[...] = jnp.where(sel, lx, ox_ref[...])
        oy_ref[...] = jnp.where(sel, ly, oy_ref[...])
        oz_ref[...] = jnp.where(sel, lz, oz_ref[...])
        dx = px - lx
        dy = py - ly
        dz = pz - lz
        d = (dx * dx + dz * dz) + dy * dy
        dist = jnp.minimum(dist_ref[...], d)
        dist_ref[...] = dist
        mx = jnp.max(dist, axis=1, keepdims=True)
        nxt = jnp.min(jnp.where(dist == mx, lanes, _N), axis=1, keepdims=True)
        return nxt.astype(jnp.int32)

    lax.fori_loop(0, _M, body, jnp.zeros((_B, 1), jnp.int32))


def _fps(px, py, pz):
    return pl.pallas_call(
        _fps_body,
        out_shape=[jax.ShapeDtypeStruct((_B, _M), jnp.float32)] * 3,
        scratch_shapes=[pltpu.VMEM((_B, _N), jnp.float32)],
    )(px, py, pz)


_MLP_BLK = 512
_MLP_STEPS = (_B * _N) // _MLP_BLK


def _mlp_body(x_ref, wt_ref, h_ref, stats_ref, acc_ref):
    i = pl.program_id(0)

    @pl.when(i == 0)
    def _():
        acc_ref[...] = jnp.zeros_like(acc_ref)

    h = jnp.dot(x_ref[...], wt_ref[...], preferred_element_type=jnp.float32)
    h_ref[...] = h
    acc_ref[0:1, :] += jnp.sum(h, axis=0, keepdims=True)
    acc_ref[1:2, :] += jnp.sum(h * h, axis=0, keepdims=True)

    @pl.when(i == _MLP_STEPS - 1)
    def _():
        stats_ref[...] = acc_ref[...]


def _mlp(xf, wt):
    return pl.pallas_call(
        _mlp_body,
        grid=(_MLP_STEPS,),
        in_specs=[
            pl.BlockSpec((_MLP_BLK, _CIN), lambda i: (i, 0)),
            pl.BlockSpec((_CIN, _COUT), lambda i: (0, 0)),
        ],
        out_specs=[
            pl.BlockSpec((_MLP_BLK, _COUT), lambda i: (i, 0)),
            pl.BlockSpec((8, _COUT), lambda i: (0, 0)),
        ],
        out_shape=[
            jax.ShapeDtypeStruct((_B * _N, _COUT), jnp.float32),
            jax.ShapeDtypeStruct((8, _COUT), jnp.float32),
        ],
        scratch_shapes=[pltpu.VMEM((8, _COUT), jnp.float32)],
    )(xf, wt)


def _bnrelu_body(h_ref, stats_ref, g_ref, b_ref, o_ref):
    cnt = jnp.float32(_B * _N)
    mean = stats_ref[0:1, :] / cnt
    var = stats_ref[1:2, :] / cnt - mean * mean
    scale = g_ref[...] / jnp.sqrt(var + _EPS)
    shift = b_ref[...] - mean * scale
    o_ref[...] = jnp.maximum(h_ref[...] * scale + shift, 0.0)


def _bnrelu(h, stats, g, b):
    return pl.pallas_call(
        _bnrelu_body,
        grid=(_MLP_STEPS,),
        in_specs=[
            pl.BlockSpec((_MLP_BLK, _COUT), lambda i: (i, 0)),
            pl.BlockSpec((8, _COUT), lambda i: (0, 0)),
            pl.BlockSpec((1, _COUT), lambda i: (0, 0)),
            pl.BlockSpec((1, _COUT), lambda i: (0, 0)),
        ],
        out_specs=pl.BlockSpec((_MLP_BLK, _COUT), lambda i: (i, 0)),
        out_shape=jax.ShapeDtypeStruct((_B * _N, _COUT), jnp.float32),
    )(h, stats, g, b)


def _knn_body(pxb_ref, pyb_ref, pzb_ref, ox_ref, oy_ref, oz_ref, ids_ref,
              d2_ref):
    b = pl.program_id(0)
    pox = ox_ref[0]
    poy = oy_ref[0]
    poz = oz_ref[0]

    rows_c = lax.broadcasted_iota(jnp.int32, (_CS, _LANES), 0)
    kio = lax.broadcasted_iota(jnp.int32, (_K, _LANES), 0)
    base = b * _N
    inf1 = jnp.full((1, _LANES), jnp.inf, jnp.float32)
    n1 = jnp.full((1, _LANES), _N, jnp.int32)

    def _combine(mn, am, cmn, cam):
        lt = cmn < mn
        eq = cmn == mn
        am2 = jnp.where(lt, cam, jnp.where(eq, jnp.minimum(am, cam), am))
        return jnp.minimum(mn, cmn), am2

    def _chunk_minarg(d2c, c):
        cmn = jnp.min(d2c, axis=0, keepdims=True)
        cand = jnp.where(d2c == cmn, rows_c + c * _CS, _N)
        cam = jnp.min(cand, axis=0, keepdims=True)
        return cmn, cam

    def fill(c, carry):
        mn, am = carry
        sl = pl.ds(c * _CS, _CS)
        dx = pxb_ref[0, sl, :] - pox
        dy = pyb_ref[0, sl, :] - poy
        dz = pzb_ref[0, sl, :] - poz
        d2 = (dx * dx + dz * dz) + dy * dy
        d2_ref[sl, :] = d2
        cmn, cam = _chunk_minarg(d2, c)
        return _combine(mn, am, cmn, cam)

    mn0, am0 = lax.fori_loop(0, _NCH, fill, (inf1, n1))
    ids0 = jnp.where(kio == 0, am0 + base, jnp.zeros((_K, _LANES), jnp.int32))

    def kpass(k, carry):
        mn, am, ids = carry

        def upass(c, carry2):
            mn2, am2 = carry2
            sl = pl.ds(c * _CS, _CS)
            hit = (rows_c + c * _CS) == am
            d2c = jnp.where(hit, jnp.inf, d2_ref[sl, :])
            d2_ref[sl, :] = d2c
            cmn, cam = _chunk_minarg(d2c, c)
            return _combine(mn2, am2, cmn, cam)

        mn, am = lax.fori_loop(0, _NCH, upass, (inf1, n1))
        ids = jnp.where(kio == k, am + base, ids)
        return mn, am, ids

    _, _, ids = lax.fori_loop(1, _K, kpass, (mn0, am0, ids0))
    ids_ref[0] = ids


def _knn(pxb, pyb, pzb, ox3, oy3, oz3):
    nblk = _M // _LANES
    po_spec = pl.BlockSpec((1, 1, _LANES), lambda b, j: (b, 0, j))
    pb_spec = pl.BlockSpec((1, _N, _LANES), lambda b, j: (b, 0, 0))
    return pl.pallas_call(
        _knn_body,
        grid=(_B, nblk),
        in_specs=[pb_spec, pb_spec, pb_spec, po_spec, po_spec, po_spec],
        out_specs=pl.BlockSpec((1, _K, _LANES), lambda b, j: (b, 0, j)),
        out_shape=jax.ShapeDtypeStruct((_B, _K, _M), jnp.int32),
        scratch_shapes=[pltpu.VMEM((_N, _LANES), jnp.float32)],
    )(pxb, pyb, pzb, ox3, oy3, oz3)


_SC_MCHUNK = 32
_SC_CHUNKS = (_B * _M) // _SC_MCHUNK
_SC_PER_W = _SC_CHUNKS // 32


def _gather_max(mlp_x, ids):
    info = plsc.get_sparse_core_info()
    nc = info.num_cores
    rows = _SC_MCHUNK * _K

    mesh = plsc.VectorSubcoreMesh(core_axis_name="c", subcore_axis_name="s")

    @functools.partial(
        pl.kernel,
        mesh=mesh,
        out_type=jax.ShapeDtypeStruct((_B * _M, _COUT), jnp.float32),
        scratch_types=[
            pltpu.VMEM((_K, 128), jnp.int32),
            pltpu.VMEM((rows,), jnp.int32),
            pltpu.VMEM((rows, _COUT), jnp.float32),
            pltpu.VMEM((_SC_MCHUNK, _COUT), jnp.float32),
            pltpu.SemaphoreType.DMA,
        ],
    )
    def k(mlp_hbm, ids_hbm, out_hbm, idx2_v, idx_v, rows_v, out_v, sem):
        wid = lax.axis_index("s") * nc + lax.axis_index("c")
        mpb = _M // _SC_MCHUNK
        grp = 128 // _SC_MCHUNK

        def group_body(gq, _):
            cid0 = wid * _SC_PER_W + gq * grp
            b = cid0 // mpb
            mo = (cid0 % mpb) * _SC_MCHUNK
            pltpu.sync_copy(ids_hbm.at[b, :, pl.ds(mo, 128)], idx2_v)

            def sub_body(s, _):
                for kk in range(_K):
                    for j in range(_SC_MCHUNK // 16):
                        idx_v[pl.ds(kk * _SC_MCHUNK + j * 16, 16)] = (
                            idx2_v[kk, pl.ds(s * _SC_MCHUNK + j * 16, 16)])
                pltpu.async_copy(mlp_hbm.at[idx_v], rows_v, sem).wait()

                def m_body(m, _):
                    def c_body(c, _):
                        sl = pl.ds(c * 16, 16)
                        acc = rows_v[m, sl]
                        for kk in range(1, _K):
                            acc = jnp.maximum(
                                acc, rows_v[kk * _SC_MCHUNK + m, sl])
                        out_v[m, sl] = acc
                        return 0

                    lax.fori_loop(0, _COUT // 16, c_body, 0)
                    return 0

                lax.fori_loop(0, _SC_MCHUNK, m_body, 0)
                pltpu.sync_copy(
                    out_v,
                    out_hbm.at[pl.ds((cid0 + s) * _SC_MCHUNK, _SC_MCHUNK)])
                return 0

            lax.fori_loop(0, grp, sub_body, 0)
            return 0

        lax.fori_loop(0, _SC_PER_W // grp, group_body, 0)

    return k(mlp_x, ids)


def kernel(x, p, W, gamma, beta):
    px = p[:, :, 0]
    py = p[:, :, 1]
    pz = p[:, :, 2]
    ox, oy, oz = _fps(px, py, pz)
    p_out = jnp.stack([ox, oy, oz], axis=-1)

    xf = x.reshape(_B * _N, _CIN)
    h, stats = _mlp(xf, W.T)
    mlp_x = _bnrelu(h, stats, gamma.reshape(1, _COUT), beta.reshape(1, _COUT))

    pxb = jnp.broadcast_to(px[:, :, None], (_B, _N, _LANES))
    pyb = jnp.broadcast_to(py[:, :, None], (_B, _N, _LANES))
    pzb = jnp.broadcast_to(pz[:, :, None], (_B, _N, _LANES))
    ids = _knn(pxb, pyb, pzb,
               ox.reshape(_B, 1, _M), oy.reshape(_B, 1, _M),
               oz.reshape(_B, 1, _M))

    y = _gather_max(mlp_x, ids)
    return y.reshape(_B, _M, _COUT), p_out


# speedup vs baseline: 1.0028x; 1.0028x over previous
import functools

import jax
import jax.numpy as jnp
from jax import lax
from jax.experimental import pallas as pl
from jax.experimental.pallas import tpu as pltpu
from jax.experimental.pallas import tpu_sc as plsc

_B, _N, _CIN, _COUT, _K = 4, 8192, 128, 128, 16
_M = _N // 4
_EPS = 1e-5

_LANES = 128
_CS = 2048
_NCH = _N // _CS


def _fps_body(px_ref, py_ref, pz_ref, ox_ref, oy_ref, oz_ref, dist_ref):
    lanes = lax.broadcasted_iota(jnp.int32, (_B, _N), 1)
    cols = lax.broadcasted_iota(jnp.int32, (_B, _M), 1)
    px = px_ref[...]
    py = py_ref[...]
    pz = pz_ref[...]
    dist_ref[...] = jnp.full((_B, _N), jnp.inf, jnp.float32)
    ox_ref[...] = jnp.zeros((_B, _M), jnp.float32)
    oy_ref[...] = jnp.zeros((_B, _M), jnp.float32)
    oz_ref[...] = jnp.zeros((_B, _M), jnp.float32)

    def body(i, cur):
        msk = lanes == cur
        lx = jnp.max(jnp.where(msk, px, -jnp.inf), axis=1, keepdims=True)
        ly = jnp.max(jnp.where(msk, py, -jnp.inf), axis=1, keepdims=True)
        lz = jnp.max(jnp.where(msk, pz, -jnp.inf), axis=1, keepdims=True)
        sel = cols == i
        ox_ref[...] = jnp.where(sel, lx, ox_ref[...])
        oy_ref[...] = jnp.where(sel, ly, oy_ref[...])
        oz_ref[...] = jnp.where(sel, lz, oz_ref[...])
        dx = px - lx
        dy = py - ly
        dz = pz - lz
        d = (dx * dx + dz * dz) + dy * dy
        dist = jnp.minimum(dist_ref[...], d)
        dist_ref[...] = dist
        mx = jnp.max(dist, axis=1, keepdims=True)
        nxt = jnp.min(jnp.where(dist == mx, lanes, _N), axis=1, keepdims=True)
        return nxt.astype(jnp.int32)

    lax.fori_loop(0, _M, body, jnp.zeros((_B, 1), jnp.int32))


def _fps(px, py, pz):
    return pl.pallas_call(
        _fps_body,
        out_shape=[jax.ShapeDtypeStruct((_B, _M), jnp.float32)] * 3,
        scratch_shapes=[pltpu.VMEM((_B, _N), jnp.float32)],
    )(px, py, pz)


_MLP_BLK = 512
_MLP_STEPS = (_B * _N) // _MLP_BLK


def _mlp_body(x_ref, wt_ref, h_ref, stats_ref, acc_ref):
    i = pl.program_id(0)

    @pl.when(i == 0)
    def _():
        acc_ref[...] = jnp.zeros_like(acc_ref)

    h = jnp.dot(x_ref[...], wt_ref[...], preferred_element_type=jnp.float32)
    h_ref[...] = h
    acc_ref[0:1, :] += jnp.sum(h, axis=0, keepdims=True)
    acc_ref[1:2, :] += jnp.sum(h * h, axis=0, keepdims=True)

    @pl.when(i == _MLP_STEPS - 1)
    def _():
        stats_ref[...] = acc_ref[...]


def _mlp(xf, wt):
    return pl.pallas_call(
        _mlp_body,
        grid=(_MLP_STEPS,),
        in_specs=[
            pl.BlockSpec((_MLP_BLK, _CIN), lambda i: (i, 0)),
            pl.BlockSpec((_CIN, _COUT), lambda i: (0, 0)),
        ],
        out_specs=[
            pl.BlockSpec((_MLP_BLK, _COUT), lambda i: (i, 0)),
            pl.BlockSpec((8, _COUT), lambda i: (0, 0)),
        ],
        out_shape=[
            jax.ShapeDtypeStruct((_B * _N, _COUT), jnp.float32),
            jax.ShapeDtypeStruct((8, _COUT), jnp.float32),
        ],
        scratch_shapes=[pltpu.VMEM((8, _COUT), jnp.float32)],
    )(xf, wt)


def _bnrelu_body(h_ref, stats_ref, g_ref, b_ref, o_ref):
    cnt = jnp.float32(_B * _N)
    mean = stats_ref[0:1, :] / cnt
    var = stats_ref[1:2, :] / cnt - mean * mean
    scale = g_ref[...] / jnp.sqrt(var + _EPS)
    shift = b_ref[...] - mean * scale
    o_ref[...] = jnp.maximum(h_ref[...] * scale + shift, 0.0)


def _bnrelu(h, stats, g, b):
    return pl.pallas_call(
        _bnrelu_body,
        grid=(_MLP_STEPS,),
        in_specs=[
            pl.BlockSpec((_MLP_BLK, _COUT), lambda i: (i, 0)),
            pl.BlockSpec((8, _COUT), lambda i: (0, 0)),
            pl.BlockSpec((1, _COUT), lambda i: (0, 0)),
            pl.BlockSpec((1, _COUT), lambda i: (0, 0)),
        ],
        out_specs=pl.BlockSpec((_MLP_BLK, _COUT), lambda i: (i, 0)),
        out_shape=jax.ShapeDtypeStruct((_B * _N, _COUT), jnp.float32),
    )(h, stats, g, b)


def _knn_body(pxb_ref, pyb_ref, pzb_ref, ox_ref, oy_ref, oz_ref, ids_ref,
              d2_ref):
    b = pl.program_id(0)
    pox = ox_ref[0]
    poy = oy_ref[0]
    poz = oz_ref[0]

    rows_c = lax.broadcasted_iota(jnp.int32, (_CS, _LANES), 0)
    kio = lax.broadcasted_iota(jnp.int32, (_K, _LANES), 0)
    base = b * _N
    inf1 = jnp.full((1, _LANES), jnp.inf, jnp.float32)
    n1 = jnp.full((1, _LANES), _N, jnp.int32)

    def _combine(mn, am, cmn, cam):
        lt = cmn < mn
        eq = cmn == mn
        am2 = jnp.where(lt, cam, jnp.where(eq, jnp.minimum(am, cam), am))
        return jnp.minimum(mn, cmn), am2

    def _chunk_minarg(d2c, c):
        cmn = jnp.min(d2c, axis=0, keepdims=True)
        cand = jnp.where(d2c == cmn, rows_c + c * _CS, _N)
        cam = jnp.min(cand, axis=0, keepdims=True)
        return cmn, cam

    def fill(c, carry):
        mn, am = carry
        sl = pl.ds(c * _CS, _CS)
        dx = pxb_ref[0, sl, :] - pox
        dy = pyb_ref[0, sl, :] - poy
        dz = pzb_ref[0, sl, :] - poz
        d2 = (dx * dx + dz * dz) + dy * dy
        d2_ref[sl, :] = d2
        cmn, cam = _chunk_minarg(d2, c)
        return _combine(mn, am, cmn, cam)

    mn0, am0 = lax.fori_loop(0, _NCH, fill, (inf1, n1))
    ids0 = jnp.where(kio == 0, am0 + base, jnp.zeros((_K, _LANES), jnp.int32))

    def kpass(k, carry):
        mn, am, ids = carry

        def upass(c, carry2):
            mn2, am2 = carry2
            sl = pl.ds(c * _CS, _CS)
            hit = (rows_c + c * _CS) == am
            d2c = jnp.where(hit, jnp.inf, d2_ref[sl, :])
            d2_ref[sl, :] = d2c
            cmn, cam = _chunk_minarg(d2c, c)
            return _combine(mn2, am2, cmn, cam)

        mn, am = lax.fori_loop(0, _NCH, upass, (inf1, n1))
        ids = jnp.where(kio == k, am + base, ids)
        return mn, am, ids

    _, _, ids = lax.fori_loop(1, _K, kpass, (mn0, am0, ids0))
    ids_ref[0] = ids


def _knn(pxb, pyb, pzb, ox3, oy3, oz3):
    nblk = _M // _LANES
    po_spec = pl.BlockSpec((1, 1, _LANES), lambda b, j: (b, 0, j))
    pb_spec = pl.BlockSpec((1, _N, _LANES), lambda b, j: (b, 0, 0))
    return pl.pallas_call(
        _knn_body,
        grid=(_B, nblk),
        in_specs=[pb_spec, pb_spec, pb_spec, po_spec, po_spec, po_spec],
        out_specs=pl.BlockSpec((1, _K, _LANES), lambda b, j: (b, 0, j)),
        out_shape=jax.ShapeDtypeStruct((_B, _K, _M), jnp.int32),
        scratch_shapes=[pltpu.VMEM((_N, _LANES), jnp.float32)],
    )(pxb, pyb, pzb, ox3, oy3, oz3)


_SC_MCHUNK = 32
_SC_CHUNKS = (_B * _M) // _SC_MCHUNK
_SC_PER_W = _SC_CHUNKS // 32


def _gather_max(mlp_x, ids):
    info = plsc.get_sparse_core_info()
    nc = info.num_cores
    rows = _SC_MCHUNK * _K

    mesh = plsc.VectorSubcoreMesh(core_axis_name="c", subcore_axis_name="s")

    @functools.partial(
        pl.kernel,
        mesh=mesh,
        out_type=jax.ShapeDtypeStruct((_B * _M, _COUT), jnp.float32),
        scratch_types=[
            pltpu.VMEM((_K, 128), jnp.int32),
            pltpu.VMEM((rows,), jnp.int32),
            pltpu.VMEM((rows, _COUT), jnp.float32),
            pltpu.VMEM((_SC_MCHUNK, _COUT), jnp.float32),
            pltpu.SemaphoreType.DMA,
        ],
    )
    def k(mlp_hbm, ids_hbm, out_hbm, idx2_v, idx_v, rows_v, out_v, sem):
        wid = lax.axis_index("s") * nc + lax.axis_index("c")
        mpb = _M // _SC_MCHUNK
        grp = 128 // _SC_MCHUNK

        def group_body(gq, _):
            cid0 = wid * _SC_PER_W + gq * grp
            b = cid0 // mpb
            mo = (cid0 % mpb) * _SC_MCHUNK
            pltpu.sync_copy(ids_hbm.at[b, :, pl.ds(mo, 128)], idx2_v)

            def sub_body(s, _):
                for kk in range(_K):
                    for j in range(_SC_MCHUNK // 16):
                        idx_v[pl.ds(kk * _SC_MCHUNK + j * 16, 16)] = (
                            idx2_v[kk, pl.ds(s * _SC_MCHUNK + j * 16, 16)])
                pltpu.async_copy(mlp_hbm.at[idx_v], rows_v, sem).wait()

                def m_body(m, _):
                    def c_body(c, _):
                        sl = pl.ds(c * 16, 16)
                        acc = rows_v[m, sl]
                        for kk in range(1, _K):
                            acc = jnp.maximum(
                                acc, rows_v[kk * _SC_MCHUNK + m, sl])
                        out_v[m, sl] = acc
                        return 0

                    lax.fori_loop(0, _COUT // 16, c_body, 0)
                    return 0

                lax.fori_loop(0, _SC_MCHUNK, m_body, 0)
                pltpu.sync_copy(
                    out_v,
                    out_hbm.at[pl.ds((cid0 + s) * _SC_MCHUNK, _SC_MCHUNK)])
                return 0

            lax.fori_loop(0, grp, sub_body, 0)
            return 0

        lax.fori_loop(0, _SC_PER_W // grp, group_body, 0)

    return k(mlp_x, ids)


def kernel(x, p, W, gamma, beta):
    px = p[:, :, 0]
    py = p[:, :, 1]
    pz = p[:, :, 2]
    ox, oy, oz = _fps(px, py, pz)
    p_out = jnp.stack([ox, oy, oz], axis=-1)

    xf = x.reshape(_B * _N, _CIN)
    h, stats = _mlp(xf, W.T)
    mlp_x = _bnrelu(h, stats, gamma.reshape(1, _COUT), beta.reshape(1, _COUT))

    pxb = jnp.broadcast_to(px[:, :, None], (_B, _N, _LANES))
    pyb = jnp.broadcast_to(py[:, :, None], (_B, _N, _LANES))
    pzb = jnp.broadcast_to(pz[:, :, None], (_B, _N, _LANES))
    ids = _knn(pxb, pyb, pzb,
               ox.reshape(_B, 1, _M), oy.reshape(_B, 1, _M),
               oz.reshape(_B, 1, _M))

    y = _gather_max(mlp_x, ids)
    return y.reshape(_B, _M, _COUT), p_out
